# baseline (speedup 1.0000x reference)
import functools

import jax
import jax.numpy as jnp
from jax import lax
from jax.experimental import pallas as pl
from jax.experimental.pallas import tpu as pltpu
from jax.experimental.pallas import tpu_sc as plsc

N_NODES = 10000
N_EDGES = 320000
D = 128

NC = 2
NS = 16
NW = NC * NS

CHUNK = 128
NCHUNKS = 78
EDGES_PER_W = NCHUNKS * CHUNK
EXTRA_BASE = NW * EDGES_PER_W
NROW = 3
NSI = 4
NDI = 4
ROWS_PER_SUB = 624
REM_ROWS = N_NODES - NS * ROWS_PER_SUB
REM_BASE = NS * ROWS_PER_SUB
ZCOPIES = ROWS_PER_SUB // CHUNK
ZREM = ROWS_PER_SUB - ZCOPIES * CHUNK


def _sc_body(x_hbm, ei_hbm, out_hbm,
             side_sh, sidx_v, didx_v, rows_v, gsem, ssem, xsem, dsem):
    c = lax.axis_index("c")
    s = lax.axis_index("s")
    wid = c * NS + s
    e0 = wid * EDGES_PER_W

    r0 = s * ROWS_PER_SUB

    @pl.when(c == 0)
    def _init_x():
        pltpu.sync_copy(x_hbm.at[pl.ds(r0, ROWS_PER_SUB)],
                        side_sh.at[pl.ds(r0, ROWS_PER_SUB)])

        @pl.when(s == 0)
        def _x_rem():
            pltpu.sync_copy(x_hbm.at[pl.ds(REM_BASE, REM_ROWS)],
                            side_sh.at[pl.ds(REM_BASE, REM_ROWS)])

    @pl.when(c == 1)
    def _init_zero():
        z16 = jnp.zeros((16,), jnp.float32)

        @pl.loop(0, CHUNK)
        def _zrow(r):
            for q in range(D // 16):
                rows_v[0, r, pl.ds(q * 16, 16)] = z16

        for j in range(ZCOPIES):
            pltpu.async_copy(rows_v.at[0],
                             side_sh.at[pl.ds(r0 + j * CHUNK, CHUNK)], ssem)
        pltpu.async_copy(rows_v.at[0].at[pl.ds(0, ZREM)],
                         side_sh.at[pl.ds(r0 + ZCOPIES * CHUNK, ZREM)], ssem)

        @pl.when(s == 0)
        def _zero_rem():
            pltpu.sync_copy(rows_v.at[0].at[pl.ds(0, REM_ROWS)],
                            side_sh.at[pl.ds(REM_BASE, REM_ROWS)])

        for j in range(ZCOPIES):
            pltpu.make_async_copy(rows_v.at[0], side_sh.at[pl.ds(0, CHUNK)],
                                  ssem).wait()
        pltpu.make_async_copy(rows_v.at[0].at[pl.ds(0, ZREM)],
                              side_sh.at[pl.ds(0, ZREM)], ssem).wait()

    plsc.subcore_barrier()

    def issue_idx(i):
        pltpu.async_copy(ei_hbm.at[0, pl.ds(e0 + i * CHUNK, CHUNK)],
                         sidx_v.at[i % NSI], xsem)
        pltpu.async_copy(ei_hbm.at[1, pl.ds(e0 + i * CHUNK, CHUNK)],
                         didx_v.at[i % NDI], dsem)

    def wait_sidx(pi):
        pltpu.make_async_copy(ei_hbm.at[0, pl.ds(0, CHUNK)], sidx_v.at[pi],
                              xsem).wait()

    def wait_didx(pi):
        pltpu.make_async_copy(ei_hbm.at[0, pl.ds(0, CHUNK)], didx_v.at[pi],
                              dsem).wait()

    def issue_gather(pi, i):
        pltpu.async_copy(x_hbm.at[sidx_v.at[pi]], rows_v.at[i % NROW], gsem)

    def wait_gather(pr):
        pltpu.make_async_copy(x_hbm.at[pl.ds(0, CHUNK)], rows_v.at[pr],
                              gsem).wait()

    for i in range(4):
        issue_idx(i)
    for i in range(3):
        wait_sidx(i % NSI)
        issue_gather(i % NSI, i)

    def step(i, refill_idx, refill_gather):
        pr, pd = i % NROW, i % NDI
        wait_gather(pr)
        wait_didx(pd)
        pltpu.sync_copy(rows_v.at[pr], side_sh.at[didx_v.at[pd]], add=True)
        if refill_idx:
            issue_idx(i + 4)
        if refill_gather:
            wait_sidx((i + 3) % NSI)
            issue_gather((i + 3) % NSI, pr)

    @pl.loop(0, 5)
    def _outer(o):
        for q in range(12):
            step(12 * o + q, True, True)

    for i in range(60, 78):
        step(i, i <= 73, i <= 74)

    @pl.when(wid % 8 == 0)
    def _extra():
        xb = EXTRA_BASE + (wid // 8) * CHUNK
        pltpu.sync_copy(ei_hbm.at[0, pl.ds(xb, CHUNK)], sidx_v.at[0])
        pltpu.sync_copy(ei_hbm.at[1, pl.ds(xb, CHUNK)], didx_v.at[0])
        pltpu.async_copy(x_hbm.at[sidx_v.at[0]], rows_v.at[0], gsem).wait()
        pltpu.sync_copy(rows_v.at[0], side_sh.at[didx_v.at[0]], add=True)

    plsc.subcore_barrier()

    o0 = c * N_NODES + s * ROWS_PER_SUB
    pltpu.sync_copy(side_sh.at[pl.ds(r0, ROWS_PER_SUB)],
                    out_hbm.at[pl.ds(o0, ROWS_PER_SUB)])

    @pl.when(s == 0)
    def _out_rem():
        pltpu.sync_copy(side_sh.at[pl.ds(REM_BASE, REM_ROWS)],
                        out_hbm.at[pl.ds(c * N_NODES + REM_BASE, REM_ROWS)])


_sc_aggregate = functools.partial(
    pl.kernel,
    out_type=jax.ShapeDtypeStruct((NC * N_NODES, D), jnp.float32),
    mesh=plsc.VectorSubcoreMesh(core_axis_name="c", subcore_axis_name="s",
                                num_cores=NC, num_subcores=NS),
    scratch_types=[
        pltpu.VMEM_SHARED((N_NODES, D), jnp.float32),
        pltpu.VMEM((NSI, CHUNK), jnp.int32),
        pltpu.VMEM((NDI, CHUNK), jnp.int32),
        pltpu.VMEM((NROW, CHUNK, D), jnp.float32),
        pltpu.SemaphoreType.DMA,
        pltpu.SemaphoreType.DMA,
        pltpu.SemaphoreType.DMA,
        pltpu.SemaphoreType.DMA,
    ],
)(_sc_body)


ROW_BLK = 2000


def _tc_body(p0_ref, p1_ref, w_ref, b_ref, o_ref):
    emb = p0_ref[...] + p1_ref[...]
    h = jnp.dot(emb, w_ref[...], preferred_element_type=jnp.float32) + b_ref[...]
    o_ref[...] = jnp.where(h >= 0, h, 0.01 * h)


def _tc_finish(ps, W, b2):
    grid = (N_NODES // ROW_BLK,)
    return pl.pallas_call(
        _tc_body,
        grid=grid,
        in_specs=[
            pl.BlockSpec((ROW_BLK, D), lambda i: (i, 0)),
            pl.BlockSpec((ROW_BLK, D), lambda i: (i + N_NODES // ROW_BLK, 0)),
            pl.BlockSpec((D, D), lambda i: (0, 0)),
            pl.BlockSpec((1, D), lambda i: (0, 0)),
        ],
        out_specs=pl.BlockSpec((ROW_BLK, D), lambda i: (i, 0)),
        out_shape=jax.ShapeDtypeStruct((N_NODES, D), jnp.float32),
    )(ps, ps, W, b2)


def kernel(x, edge_index, W, b):
    ps = _sc_aggregate(x, edge_index.astype(jnp.int32))
    return _tc_finish(ps, W, b.reshape(1, D))


# speedup vs baseline: 1.0203x; 1.0203x over previous
import functools

import jax
import jax.numpy as jnp
from jax import lax
from jax.experimental import pallas as pl
from jax.experimental.pallas import tpu as pltpu
from jax.experimental.pallas import tpu_sc as plsc

N_NODES = 10000
N_EDGES = 320000
D = 128

NC = 2
NS = 16
NW = NC * NS

CHUNK = 128
NCHUNKS = 78
EDGES_PER_W = NCHUNKS * CHUNK
EXTRA_BASE = NW * EDGES_PER_W
NROW = 3
NSI = 4
NDI = 4
ROWS_PER_SUB = 624
REM_ROWS = N_NODES - NS * ROWS_PER_SUB
REM_BASE = NS * ROWS_PER_SUB
ZCOPIES = ROWS_PER_SUB // CHUNK
ZREM = ROWS_PER_SUB - ZCOPIES * CHUNK


def _sc_body(x_hbm, ei_hbm, out_hbm,
             side_sh, sidx_v, didx_v, rows_v, gsem, ssem, xsem, dsem):
    c = lax.axis_index("c")
    s = lax.axis_index("s")
    wid = c * NS + s
    e0 = wid * EDGES_PER_W

    z16 = jnp.zeros((16,), jnp.float32)

    @pl.loop(0, CHUNK)
    def _zrow(r):
        for q in range(D // 16):
            rows_v[0, r, pl.ds(q * 16, 16)] = z16

    r0 = s * ROWS_PER_SUB
    for j in range(ZCOPIES):
        pltpu.async_copy(rows_v.at[0],
                         side_sh.at[pl.ds(r0 + j * CHUNK, CHUNK)], ssem)
    pltpu.async_copy(rows_v.at[0].at[pl.ds(0, ZREM)],
                     side_sh.at[pl.ds(r0 + ZCOPIES * CHUNK, ZREM)], ssem)

    @pl.when(s == 0)
    def _zero_rem():
        pltpu.sync_copy(rows_v.at[0].at[pl.ds(0, REM_ROWS)],
                        side_sh.at[pl.ds(REM_BASE, REM_ROWS)])

    for j in range(ZCOPIES):
        pltpu.make_async_copy(rows_v.at[0], side_sh.at[pl.ds(0, CHUNK)],
                              ssem).wait()
    pltpu.make_async_copy(rows_v.at[0].at[pl.ds(0, ZREM)],
                          side_sh.at[pl.ds(0, ZREM)], ssem).wait()
    plsc.subcore_barrier()

    def issue_idx(i):
        pltpu.async_copy(ei_hbm.at[0, pl.ds(e0 + i * CHUNK, CHUNK)],
                         sidx_v.at[i % NSI], xsem)
        pltpu.async_copy(ei_hbm.at[1, pl.ds(e0 + i * CHUNK, CHUNK)],
                         didx_v.at[i % NDI], dsem)

    def wait_sidx(pi):
        pltpu.make_async_copy(ei_hbm.at[0, pl.ds(0, CHUNK)], sidx_v.at[pi],
                              xsem).wait()

    def wait_didx(pi):
        pltpu.make_async_copy(ei_hbm.at[0, pl.ds(0, CHUNK)], didx_v.at[pi],
                              dsem).wait()

    def issue_gather(pi, i):
        pltpu.async_copy(x_hbm.at[sidx_v.at[pi]], rows_v.at[i % NROW], gsem)

    def wait_gather(pr):
        pltpu.make_async_copy(x_hbm.at[pl.ds(0, CHUNK)], rows_v.at[pr],
                              gsem).wait()

    for i in range(4):
        issue_idx(i)
    for i in range(3):
        wait_sidx(i % NSI)
        issue_gather(i % NSI, i)

    def step(i, refill_idx, refill_gather):
        pr, pd = i % NROW, i % NDI
        wait_gather(pr)
        wait_didx(pd)
        pltpu.sync_copy(rows_v.at[pr], side_sh.at[didx_v.at[pd]], add=True)
        if refill_idx:
            issue_idx(i + 4)
        if refill_gather:
            wait_sidx((i + 3) % NSI)
            issue_gather((i + 3) % NSI, pr)

    @pl.loop(0, 5)
    def _outer(o):
        for q in range(12):
            step(12 * o + q, True, True)

    for i in range(60, 78):
        step(i, i <= 73, i <= 74)

    @pl.when(wid % 8 == 0)
    def _extra():
        xb = EXTRA_BASE + (wid // 8) * CHUNK
        pltpu.sync_copy(ei_hbm.at[0, pl.ds(xb, CHUNK)], sidx_v.at[0])
        pltpu.sync_copy(ei_hbm.at[1, pl.ds(xb, CHUNK)], didx_v.at[0])
        pltpu.async_copy(x_hbm.at[sidx_v.at[0]], rows_v.at[0], gsem).wait()
        pltpu.sync_copy(rows_v.at[0], side_sh.at[didx_v.at[0]], add=True)

    plsc.subcore_barrier()

    o0 = c * N_NODES + s * ROWS_PER_SUB
    pltpu.sync_copy(side_sh.at[pl.ds(r0, ROWS_PER_SUB)],
                    out_hbm.at[pl.ds(o0, ROWS_PER_SUB)])

    @pl.when(s == 0)
    def _out_rem():
        pltpu.sync_copy(side_sh.at[pl.ds(REM_BASE, REM_ROWS)],
                        out_hbm.at[pl.ds(c * N_NODES + REM_BASE, REM_ROWS)])


_sc_aggregate = functools.partial(
    pl.kernel,
    out_type=jax.ShapeDtypeStruct((NC * N_NODES, D), jnp.float32),
    mesh=plsc.VectorSubcoreMesh(core_axis_name="c", subcore_axis_name="s",
                                num_cores=NC, num_subcores=NS),
    scratch_types=[
        pltpu.VMEM_SHARED((N_NODES, D), jnp.float32),
        pltpu.VMEM((NSI, CHUNK), jnp.int32),
        pltpu.VMEM((NDI, CHUNK), jnp.int32),
        pltpu.VMEM((NROW, CHUNK, D), jnp.float32),
        pltpu.SemaphoreType.DMA,
        pltpu.SemaphoreType.DMA,
        pltpu.SemaphoreType.DMA,
        pltpu.SemaphoreType.DMA,
    ],
)(_sc_body)


ROW_BLK = 2000


def _tc_body(x_ref, p0_ref, p1_ref, w_ref, b_ref, o_ref):
    emb = x_ref[...] + p0_ref[...] + p1_ref[...]
    h = jnp.dot(emb, w_ref[...], preferred_element_type=jnp.float32) + b_ref[...]
    o_ref[...] = jnp.where(h >= 0, h, 0.01 * h)


def _tc_finish(x, ps, W, b2):
    grid = (N_NODES // ROW_BLK,)
    return pl.pallas_call(
        _tc_body,
        grid=grid,
        in_specs=[
            pl.BlockSpec((ROW_BLK, D), lambda i: (i, 0)),
            pl.BlockSpec((ROW_BLK, D), lambda i: (i, 0)),
            pl.BlockSpec((ROW_BLK, D), lambda i: (i + N_NODES // ROW_BLK, 0)),
            pl.BlockSpec((D, D), lambda i: (0, 0)),
            pl.BlockSpec((1, D), lambda i: (0, 0)),
        ],
        out_specs=pl.BlockSpec((ROW_BLK, D), lambda i: (i, 0)),
        out_shape=jax.ShapeDtypeStruct((N_NODES, D), jnp.float32),
    )(x, ps, ps, W, b2)


def kernel(x, edge_index, W, b):
    ps = _sc_aggregate(x, edge_index.astype(jnp.int32))
    return _tc_finish(x, ps, W, b.reshape(1, D))
